# baseline (speedup 1.0000x reference)
import functools

import jax
import jax.numpy as jnp
import numpy as np
from jax import lax
from jax.experimental import pallas as pl
from jax.experimental.pallas import tpu as pltpu
from jax.experimental.pallas import tpu_sc as plsc

D_MODEL = 64
_NC = 2
_NS = 16
_NW = _NC * _NS
_W = 256
_LOG2W = 8
_CAP = 256
_NSTAGE = 256


def _pe_row_np(pos):
    j = np.arange(D_MODEL, dtype=np.float32)
    angle = np.float32(pos) / np.power(np.float32(10000.0), 2.0 * j / D_MODEL)
    pe = np.where(np.arange(D_MODEL) % 2 == 0, np.sin(angle), np.cos(angle))
    return [float(v) for v in pe.astype(np.float32)]


@functools.lru_cache(maxsize=None)
def _make_kernel(B, V):
    per_w = B // _NW
    nv = per_w // 16
    K = (V + _W - 1) // _W
    KP = ((K + 127) // 128) * 128
    piece = 128
    npiece = KP // piece
    pseg = ((B + 2 * 8 * K + _CAP + 2047) // 2048) * 2048
    out_rows = B + 16
    dump = B
    chunks_per_sc = (K + _NS - 1) // _NS
    tail_shift = (V - 128) - (K - 1) * _W
    pe_consts = _pe_row_np(200)

    mesh = plsc.VectorSubcoreMesh(core_axis_name="c", subcore_axis_name="s")

    @functools.partial(
        pl.kernel,
        mesh=mesh,
        compiler_params=pltpu.CompilerParams(
            use_tc_tiling_on_sc=True, needs_layout_passes=False),
        out_type=jax.ShapeDtypeStruct((out_rows, 128), jnp.float32),
        scratch_types=[
            pltpu.VMEM((per_w,), jnp.int32),
            pltpu.VMEM((KP,), jnp.int32),
            pltpu.VMEM((_NW, piece), jnp.int32),
            pltpu.VMEM((KP,), jnp.int32),
            pltpu.VMEM((KP,), jnp.int32),
            pltpu.VMEM((KP,), jnp.int32),
            pltpu.VMEM((KP,), jnp.int32),
            pltpu.VMEM((KP,), jnp.int32),
            pltpu.VMEM((per_w // 128, 128), jnp.int32),
            pltpu.VMEM((per_w // 128, 128), jnp.int32),
            pltpu.VMEM((D_MODEL, _W), jnp.float32),
            pltpu.VMEM((_CAP,), jnp.int32),
            pltpu.VMEM((_NSTAGE, 128), jnp.float32),
            pltpu.VMEM((_NSTAGE // 128, 128), jnp.int32),
            pltpu.VMEM((16,), jnp.int32),
            pltpu.VMEM_SHARED((pseg,), jnp.int32),
            pltpu.HBM((_NW, KP), jnp.int32),
            pltpu.HBM((_NC, 16), jnp.int32),
            pltpu.SemaphoreType.DMA,
            pltpu.SemaphoreType.DMA,
        ],
    )
    def body(idx_hbm, tabT_hbm, tail_hbm, out_hbm, idx_v, hist_v,
             piece_v, totals_v, halfa_v, starts_v, mid_v, cur_v, slots_v,
             vals_v, chunk_v, pair_v, stage_v, posr_v, flag_v, shared_sorted,
             hist_hbm, flags_hbm, sem, sem2):
        cid = lax.axis_index("c")
        sid = lax.axis_index("s")
        w = cid * _NS + sid
        base = w * per_w
        iota = lax.iota(jnp.int32, 16)
        zeros16 = jnp.zeros((16,), jnp.int32)

        @pl.when(sid == 0)
        def _():
            flag_v[...] = zeros16
            pltpu.sync_copy(flag_v, flags_hbm.at[cid])

        def mailbox(round_no):
            plsc.subcore_barrier()

            @pl.when(sid == 0)
            def _():
                flag_v[...] = jnp.full((16,), round_no, jnp.int32)
                pltpu.sync_copy(flag_v, flags_hbm.at[1 - cid])

                def poll_cond(v):
                    return v < round_no

                def poll_body(v):
                    pltpu.sync_copy(flags_hbm.at[cid], flag_v)
                    got = plsc.load_gather(flag_v, [zeros16])
                    return lax.reduce_max(got, axes=(0,))

                lax.while_loop(poll_cond, poll_body, jnp.int32(0))

            plsc.subcore_barrier()

        pltpu.sync_copy(
            idx_hbm.at[pl.ds(pl.multiple_of(base, 8), per_w)], idx_v)

        def zero_hist(i, _):
            hist_v[pl.ds(i * 16, 16)] = zeros16
            return 0

        lax.fori_loop(0, KP // 16, zero_hist, 0)

        def hist_step(i, _):
            tok = idx_v[pl.ds(i * 16, 16)]
            key = lax.shift_right_logical(tok, _LOG2W)
            sk, _ = plsc.sort_key_val(key, tok)
            cnt, last = plsc.scan_count(sk)
            plsc.addupdate_scatter(hist_v, [sk], cnt, mask=last)
            return 0

        lax.fori_loop(0, nv, hist_step, 0)

        pltpu.sync_copy(hist_v, hist_hbm.at[w])
        mailbox(1)

        def piece_step(p, _):
            pltpu.sync_copy(hist_hbm.at[:, pl.ds(p * piece, piece)], piece_v)

            def col_block(j, _):
                acc = zeros16
                mine = zeros16
                acc_a = zeros16
                for wi in range(_NW):
                    row = piece_v[wi, pl.ds(j * 16, 16)]
                    acc = acc + row
                    mine = mine + jnp.where(w > wi, row, 0)
                    if wi == _NS - 1:
                        acc_a = acc
                o = p * piece + j * 16
                totals_v[pl.ds(o, 16)] = acc
                halfa_v[pl.ds(o, 16)] = acc_a
                cur_v[pl.ds(o, 16)] = mine
                return 0

            lax.fori_loop(0, piece // 16, col_block, 0)
            return 0

        lax.fori_loop(0, npiece, piece_step, 0)

        def prefix_step(i, carry):
            t = totals_v[pl.ds(i * 16, 16)]
            a = halfa_v[pl.ds(i * 16, 16)]
            pad_a = jnp.bitwise_and(a + 7, -8)
            pad_b = jnp.bitwise_and(t - a + 7, -8)
            tp = pad_a + pad_b
            inc = plsc.cumsum(tp)
            excl = inc - tp + carry
            starts_v[pl.ds(i * 16, 16)] = excl
            mid_v[pl.ds(i * 16, 16)] = excl + pad_a
            m = cur_v[pl.ds(i * 16, 16)]
            cur_v[pl.ds(i * 16, 16)] = jnp.where(
                w < _NS, excl + m, excl + pad_a + m - a)
            return carry + lax.reduce_sum(tp, axes=(0,))

        lax.fori_loop(0, KP // 16, prefix_step, jnp.int32(0))

        def scatter_step(j, _):
            for u in range(8):
                i = j * 8 + u
                tok = idx_v[pl.ds(i * 16, 16)]
                key = lax.shift_right_logical(tok, _LOG2W)
                pos = base + i * 16 + iota
                rec = jnp.bitwise_or(
                    lax.shift_left(pos, 8), jnp.bitwise_and(tok, _W - 1))
                sk, sv = plsc.sort_key_val(key, rec)
                cnt, last = plsc.scan_count(sk)
                off = plsc.load_gather(cur_v, [sk])
                slots_v[j, pl.ds(u * 16, 16)] = off + cnt - 1
                vals_v[j, pl.ds(u * 16, 16)] = sv
                plsc.store_scatter(cur_v, [sk], off + cnt, mask=last)
            return 0

        lax.fori_loop(0, per_w // 128, scatter_step, 0)

        def fire(j, _):
            pltpu.async_copy(
                vals_v.at[j], shared_sorted.at[slots_v.at[j]], sem2)
            return 0

        lax.fori_loop(0, per_w // 128, fire, 0)

        def drain(j, _):
            pltpu.make_async_copy(
                vals_v.at[j], shared_sorted.at[slots_v.at[j]], sem2).wait()
            return 0

        lax.fori_loop(0, per_w // 128, drain, 0)

        plsc.subcore_barrier()

        def serve_seg(begin, count, coladd):
            def batch_cond(done):
                return done < count

            def batch_body(done):
                pltpu.sync_copy(
                    shared_sorted.at[pl.ds(
                        pl.multiple_of(begin + done, 8), _CAP)],
                    pair_v)
                m = jnp.minimum(count - done, _CAP)
                ng = lax.shift_right_logical(m + 15, 4)

                def fill_dump(q, _):
                    posr_v[lax.shift_right_logical(q, 3),
                           pl.ds(jnp.bitwise_and(q, 7) * 16, 16)] = (
                               zeros16 + dump)
                    return 0

                lax.fori_loop(0, _NSTAGE // 16, fill_dump, 0)

                def group(g, _):
                    rec = pair_v[pl.ds(g * 16, 16)]
                    valid = (g * 16 + iota) < m
                    pos = jnp.where(
                        valid, lax.shift_right_logical(rec, 8), dump)
                    col = jnp.bitwise_and(rec, _W - 1) + coladd
                    rows = g * 16 + iota
                    for d in range(D_MODEL):
                        dvec = jnp.full((16,), d, jnp.int32)
                        vals = plsc.load_gather(chunk_v, [dvec, col])
                        plsc.store_scatter(
                            stage_v, [rows, dvec], vals + pe_consts[d])
                    posr_v[lax.shift_right_logical(g, 3),
                           pl.ds(jnp.bitwise_and(g, 7) * 16, 16)] = pos
                    return 0

                lax.fori_loop(0, ng, group, 0)

                for j2 in range(_NSTAGE // 128):
                    @pl.when(j2 * 128 < m)
                    def _(j2=j2):
                        pltpu.async_copy(
                            stage_v.at[pl.ds(j2 * 128, 128)],
                            out_hbm.at[posr_v.at[j2]], sem)
                        pltpu.make_async_copy(
                            stage_v.at[pl.ds(j2 * 128, 128)],
                            out_hbm.at[posr_v.at[j2]], sem).wait()
                return done + _CAP

            lax.while_loop(batch_cond, batch_body, jnp.int32(0))

        def serve_chunk(ci, _):
            c = sid + ci * _NS

            @pl.when(c < K)
            def _():
                @pl.when(c < K - 1)
                def _():
                    pltpu.sync_copy(
                        tabT_hbm.at[:, pl.ds(c * _W, _W)], chunk_v)

                @pl.when(c == K - 1)
                def _():
                    pltpu.sync_copy(tail_hbm, chunk_v.at[:, pl.ds(0, 128)])

                coladd = jnp.where(c == K - 1, -tail_shift, 0)
                cvec = zeros16 + c
                n_t = lax.reduce_max(
                    plsc.load_gather(totals_v, [cvec]), axes=(0,))
                n_a = lax.reduce_max(
                    plsc.load_gather(halfa_v, [cvec]), axes=(0,))
                st = lax.reduce_max(
                    plsc.load_gather(starts_v, [cvec]), axes=(0,))
                md = lax.reduce_max(
                    plsc.load_gather(mid_v, [cvec]), axes=(0,))
                begin = jnp.where(cid == 0, st, md)
                count = jnp.where(cid == 0, n_a, n_t - n_a)
                serve_seg(begin, count, coladd)

            return 0

        lax.fori_loop(0, chunks_per_sc, serve_chunk, 0)

    return body


def kernel(x, table):
    Bb, Ls = x.shape
    V, D = table.shape
    tabT = table.T
    out = _make_kernel(Bb * Ls, V)(x.reshape(-1), tabT, tabT[:, V - 128:])
    return out[:Bb * Ls, :D].reshape(Bb, Ls, D)


# speedup vs baseline: 21.0296x; 21.0296x over previous
import functools

import jax
import jax.numpy as jnp
import numpy as np
from jax import lax
from jax.experimental import pallas as pl
from jax.experimental.pallas import tpu as pltpu
from jax.experimental.pallas import tpu_sc as plsc

D_MODEL = 64
MAX_SEQ_LEN = 256

_NC = 2
_NS = 16
_NW = _NC * _NS


def _pe_row(pos):
    j = jnp.arange(D_MODEL, dtype=jnp.float32)
    angle = pos / jnp.power(10000.0, 2.0 * j / D_MODEL)
    even = (jnp.arange(D_MODEL) % 2 == 0)
    return jnp.where(even, jnp.sin(angle), jnp.cos(angle))


@functools.lru_cache(maxsize=None)
def _make_kernel(B, V):
    assert B % _NW == 0
    per_w = B // _NW
    C = 640
    assert per_w % C == 0
    n_chunks = per_w // C

    mesh = plsc.VectorSubcoreMesh(core_axis_name="c", subcore_axis_name="s")

    @functools.partial(
        pl.kernel,
        mesh=mesh,
        compiler_params=pltpu.CompilerParams(
            use_tc_tiling_on_sc=True, needs_layout_passes=False),
        out_type=jax.ShapeDtypeStruct((B, 128), jnp.float32),
        scratch_types=[
            pltpu.VMEM((per_w,), jnp.int32),
            pltpu.VMEM((per_w,), jnp.int32),
            pltpu.VMEM((C, 128), jnp.float32),
            pltpu.VMEM((D_MODEL,), jnp.float32),
            pltpu.SemaphoreType.DMA,
        ],
    )
    def body(idx_hbm, tab2_hbm, pe_hbm, out_hbm, idx_v, idx2_v, rows_v,
             pe_v, sem):
        cid = lax.axis_index("c")
        sid = lax.axis_index("s")
        w = sid * _NC + cid
        base = w * per_w
        iota = lax.iota(jnp.int32, 16)
        zeros16 = jnp.zeros((16,), jnp.int32)

        pltpu.sync_copy(pe_hbm, pe_v)
        pltpu.sync_copy(
            idx_hbm.at[pl.ds(pl.multiple_of(base, 8), per_w)], idx_v)

        def halve(i, _):
            idx2_v[pl.ds(i * 16, 16)] = lax.shift_right_logical(
                idx_v[pl.ds(i * 16, 16)], 1)
            return 0

        lax.fori_loop(0, per_w // 16, halve, 0)

        for g in range(n_chunks):
            pltpu.async_copy(
                tab2_hbm.at[idx2_v.at[pl.ds(g * C, C)]], rows_v, sem
            ).wait()

            def fixup(j, _):
                r16 = j * 16 + iota
                tok = idx_v[pl.ds(g * C + j * 16, 16)]
                odd = jnp.bitwise_and(tok, 1) > 0

                def dstep(dd, _):
                    dvec = zeros16 + dd
                    hi = plsc.load_gather(rows_v, [r16, dvec + D_MODEL])
                    lo = plsc.load_gather(rows_v, [r16, dvec])
                    pe = plsc.load_gather(pe_v, [dvec])
                    plsc.store_scatter(
                        rows_v, [r16, dvec],
                        jnp.where(odd, hi, lo) + pe)
                    return 0

                lax.fori_loop(0, D_MODEL, dstep, 0, unroll=8)
                return 0

            lax.fori_loop(0, C // 16, fixup, 0)
            pltpu.sync_copy(
                rows_v, out_hbm.at[pl.ds(base + g * C, C)])

    return body


def kernel(x, table):
    Bb, Ls = x.shape
    V, D = table.shape
    tab2 = table.reshape(V // 2, 2 * D)
    pe = _pe_row(Ls)
    out = _make_kernel(Bb * Ls, V)(x.reshape(-1), tab2, pe)
    return out[:, :D].reshape(Bb, Ls, D)


# speedup vs baseline: 39.3461x; 1.8710x over previous
import functools

import jax
import jax.numpy as jnp
import numpy as np
from jax import lax
from jax.experimental import pallas as pl
from jax.experimental.pallas import tpu as pltpu
from jax.experimental.pallas import tpu_sc as plsc

D_MODEL = 64
MAX_SEQ_LEN = 256

_NC = 2
_NS = 16
_NW = _NC * _NS


def _pe_row(pos):
    j = jnp.arange(D_MODEL, dtype=jnp.float32)
    angle = pos / jnp.power(10000.0, 2.0 * j / D_MODEL)
    even = (jnp.arange(D_MODEL) % 2 == 0)
    return jnp.where(even, jnp.sin(angle), jnp.cos(angle))


@functools.lru_cache(maxsize=None)
def _make_kernel(B, V):
    assert B % _NW == 0
    per_w = B // _NW
    C = 640
    assert per_w % C == 0
    n_chunks = per_w // C

    mesh = plsc.VectorSubcoreMesh(core_axis_name="c", subcore_axis_name="s")

    @functools.partial(
        pl.kernel,
        mesh=mesh,
        compiler_params=pltpu.CompilerParams(
            use_tc_tiling_on_sc=True, needs_layout_passes=False),
        out_type=jax.ShapeDtypeStruct((B, 128), jnp.float32),
        scratch_types=[
            pltpu.VMEM((per_w,), jnp.int32),
            pltpu.VMEM((per_w,), jnp.int32),
            pltpu.VMEM((C, 128), jnp.float32),
            pltpu.VMEM((D_MODEL,), jnp.float32),
            pltpu.SemaphoreType.DMA,
        ],
    )
    def body(idx_hbm, tab2_hbm, pe_hbm, out_hbm, idx_v, idx2_v, rows_v,
             pe_v, sem):
        cid = lax.axis_index("c")
        sid = lax.axis_index("s")
        w = sid * _NC + cid
        base = w * per_w
        iota = lax.iota(jnp.int32, 16)
        zeros16 = jnp.zeros((16,), jnp.int32)

        pltpu.sync_copy(pe_hbm, pe_v)
        pltpu.sync_copy(
            idx_hbm.at[pl.ds(pl.multiple_of(base, 8), per_w)], idx_v)

        def halve(i, _):
            idx2_v[pl.ds(i * 16, 16)] = lax.shift_right_logical(
                idx_v[pl.ds(i * 16, 16)], 1)
            return 0

        lax.fori_loop(0, per_w // 16, halve, 0)

        pe_regs = [pe_v[pl.ds(k * 16, 16)] for k in range(D_MODEL // 16)]

        for g in range(n_chunks):
            pltpu.async_copy(
                tab2_hbm.at[idx2_v.at[pl.ds(g * C, C)]], rows_v, sem
            ).wait()

            def fixup(rr, _):
                tokv = plsc.load_gather(idx_v, [zeros16 + (g * C + rr)])
                odd = jnp.bitwise_and(tokv, 1) > 0
                for k in range(D_MODEL // 16):
                    lo = rows_v[rr, pl.ds(k * 16, 16)]
                    hi = rows_v[rr, pl.ds(D_MODEL + k * 16, 16)]
                    rows_v[rr, pl.ds(k * 16, 16)] = (
                        jnp.where(odd, hi, lo) + pe_regs[k])
                return 0

            lax.fori_loop(0, C, fixup, 0, unroll=2)
            pltpu.sync_copy(
                rows_v, out_hbm.at[pl.ds(base + g * C, C)])

    return body


def kernel(x, table):
    Bb, Ls = x.shape
    V, D = table.shape
    tab2 = table.reshape(V // 2, 2 * D)
    pe = _pe_row(Ls)
    out = _make_kernel(Bb * Ls, V)(x.reshape(-1), tab2, pe)
    return out[:, :D].reshape(Bb, Ls, D)
